# baseline (speedup 1.0000x reference)
import functools

import jax
import jax.numpy as jnp
from jax import lax
from jax.experimental import pallas as pl
from jax.experimental.pallas import tpu as pltpu
from jax.experimental.pallas import tpu_sc as plsc

_B = 16384
_E = 16
_F = 128 // _E


@jax.jit
def _sc_gather3(cat_idx, store_idx, pa_idx, cat_r, store_r, pa_r):
    info = plsc.get_sparse_core_info()
    nc, ns = info.num_cores, info.num_subcores
    nw = nc * ns
    bpw = _B // nw

    mesh = plsc.VectorSubcoreMesh(core_axis_name="c", subcore_axis_name="s")

    @functools.partial(
        pl.kernel,
        mesh=mesh,
        out_type=[jax.ShapeDtypeStruct((_B, 128), jnp.float32)] * 3,
        scratch_types=[
            pltpu.VMEM((bpw,), jnp.int32),
            pltpu.VMEM((bpw, 128), jnp.float32),
            pltpu.SemaphoreType.DMA,
        ],
    )
    def gather_kernel(cat_i, store_i, pa_i, cat_t, store_t, pa_t,
                      cat_o, store_o, pa_o, idx_v, rows_v, sem):
        wid = lax.axis_index("s") * nc + lax.axis_index("c")
        base = wid * bpw
        for i_hbm, t_hbm, o_hbm in ((cat_i, cat_t, cat_o),
                                    (store_i, store_t, store_o),
                                    (pa_i, pa_t, pa_o)):
            pltpu.sync_copy(i_hbm.at[pl.ds(base, bpw)], idx_v)
            for j in range(bpw // 16):
                sl = pl.ds(j * 16, 16)
                idx_v[sl] = idx_v[sl] >> 3
            pltpu.async_copy(t_hbm.at[idx_v], rows_v, sem).wait()
            pltpu.sync_copy(rows_v, o_hbm.at[pl.ds(base, bpw)])

    return gather_kernel(cat_idx, store_idx, pa_idx, cat_r, store_r, pa_r)


def _dense_body(ci_ref, si_ref, pi_ref, cat_g, store_g, pa_g, num_ref,
                title_ref, wn_ref, bn_ref, wt_ref, bt_ref,
                wc_ref, ws_ref, wp_ref, wo_ref, bo_ref, out_ref):
    shape = cat_g.shape
    lane_grp = lax.broadcasted_iota(jnp.int32, shape, 1) >> 4
    f32 = jnp.float32

    def pick(idx_ref, g_ref, w_ref):
        mask = lane_grp == (idx_ref[...] & (_F - 1))
        g = jnp.where(mask, g_ref[...], 0.0)
        return jnp.dot(g, w_ref[...], preferred_element_type=f32)

    acc = pick(ci_ref, cat_g, wc_ref)
    acc += pick(si_ref, store_g, ws_ref)
    acc += pick(pi_ref, pa_g, wp_ref)
    wo = wo_ref[...]
    nf = jnp.dot(num_ref[...], wn_ref[...], preferred_element_type=f32)
    nf += bn_ref[...]
    acc += jnp.dot(nf, wo[48:64, :], preferred_element_type=f32)
    te = jnp.dot(title_ref[...], wt_ref[...], preferred_element_type=f32)
    te += bt_ref[...]
    acc += jnp.dot(te, wo[64:128, :], preferred_element_type=f32)
    out_ref[...] = acc + bo_ref[...]


@jax.jit
def _tc_dense(ci, si, pi, cat_g, store_g, pa_g, num_pad, title,
              WnT, bn2, WtT, bt2, WC, WS, WP, WoT, bo2):
    R = 2048
    grid = (_B // R,)
    row_blk = lambda i: (i, 0)
    full = lambda i: (0, 0)
    return pl.pallas_call(
        _dense_body,
        grid=grid,
        in_specs=[
            pl.BlockSpec((R, 1), row_blk),
            pl.BlockSpec((R, 1), row_blk),
            pl.BlockSpec((R, 1), row_blk),
            pl.BlockSpec((R, 128), row_blk),
            pl.BlockSpec((R, 128), row_blk),
            pl.BlockSpec((R, 128), row_blk),
            pl.BlockSpec((R, 8), row_blk),
            pl.BlockSpec((R, 384), row_blk),
            pl.BlockSpec((8, 16), full),
            pl.BlockSpec((1, 16), full),
            pl.BlockSpec((384, 64), full),
            pl.BlockSpec((1, 64), full),
            pl.BlockSpec((128, 128), full),
            pl.BlockSpec((128, 128), full),
            pl.BlockSpec((128, 128), full),
            pl.BlockSpec((128, 128), full),
            pl.BlockSpec((1, 128), full),
        ],
        out_specs=pl.BlockSpec((R, 128), row_blk),
        out_shape=jax.ShapeDtypeStruct((_B, 128), jnp.float32),
        compiler_params=pltpu.CompilerParams(
            dimension_semantics=("arbitrary",),
        ),
    )(ci, si, pi, cat_g, store_g, pa_g, num_pad, title,
      WnT, bn2, WtT, bt2, WC, WS, WP, WoT, bo2)


def kernel(category, store, parent_asin, numeric_features, title_embedding,
           cat_table, store_table, pa_table, Wn, bn, Wt, bt, Wo, bo):
    ci = category.astype(jnp.int32)
    si = store.astype(jnp.int32)
    pi = parent_asin.astype(jnp.int32)
    cat_g, store_g, pa_g = _sc_gather3(
        ci, si, pi,
        cat_table.reshape(-1, 128), store_table.reshape(-1, 128),
        pa_table.reshape(-1, 128))
    num_pad = jnp.pad(numeric_features, ((0, 0), (0, 5)))
    WnT = jnp.pad(Wn.T, ((0, 5), (0, 0)))
    WoT = Wo.T
    WC = jnp.tile(WoT[0:16], (_F, 1))
    WS = jnp.tile(WoT[16:32], (_F, 1))
    WP = jnp.tile(WoT[32:48], (_F, 1))
    return _tc_dense(
        ci.reshape(-1, 1), si.reshape(-1, 1), pi.reshape(-1, 1),
        cat_g, store_g, pa_g, num_pad, title_embedding,
        WnT, bn.reshape(1, 16), Wt.T, bt.reshape(1, 64),
        WC, WS, WP, WoT, bo.reshape(1, 128))


# speedup vs baseline: 1.0491x; 1.0491x over previous
import functools

import jax
import jax.numpy as jnp
from jax import lax
from jax.experimental import pallas as pl
from jax.experimental.pallas import tpu as pltpu
from jax.experimental.pallas import tpu_sc as plsc

_B = 16384
_E = 16


@jax.jit
def _sc_gather3(cat_idx, store_idx, pa_idx, cat_t, store_t, pa_t):
    info = plsc.get_sparse_core_info()
    nc, ns = info.num_cores, info.num_subcores
    nw = nc * ns
    bpw = _B // nw

    mesh = plsc.VectorSubcoreMesh(core_axis_name="c", subcore_axis_name="s")

    @functools.partial(
        pl.kernel,
        mesh=mesh,
        out_type=[jax.ShapeDtypeStruct((_B, _E), jnp.float32)] * 3,
        compiler_params=pltpu.CompilerParams(use_tc_tiling_on_sc=False),
        scratch_types=[
            pltpu.VMEM((bpw,), jnp.int32),
            pltpu.VMEM((bpw, _E), jnp.float32),
            pltpu.SemaphoreType.DMA,
        ],
    )
    def gather_kernel(cat_i, store_i, pa_i, cat_tbl, store_tbl, pa_tbl,
                      cat_o, store_o, pa_o, idx_v, rows_v, sem):
        wid = lax.axis_index("s") * nc + lax.axis_index("c")
        base = wid * bpw
        for i_hbm, t_hbm, o_hbm in ((cat_i, cat_tbl, cat_o),
                                    (store_i, store_tbl, store_o),
                                    (pa_i, pa_tbl, pa_o)):
            pltpu.sync_copy(i_hbm.at[pl.ds(base, bpw)], idx_v)
            pltpu.async_copy(t_hbm.at[idx_v], rows_v, sem).wait()
            pltpu.sync_copy(rows_v, o_hbm.at[pl.ds(base, bpw)])

    return gather_kernel(cat_idx, store_idx, pa_idx, cat_t, store_t, pa_t)


def _dense_body(cat_g, store_g, pa_g, num_ref, title_ref,
                wn_ref, bn_ref, wt_ref, bt_ref, wo_ref, bo_ref, out_ref):
    f32 = jnp.float32
    wo = wo_ref[...]
    acc = jnp.dot(cat_g[...], wo[0:16, :], preferred_element_type=f32)
    acc += jnp.dot(store_g[...], wo[16:32, :], preferred_element_type=f32)
    acc += jnp.dot(pa_g[...], wo[32:48, :], preferred_element_type=f32)
    nf = jnp.dot(num_ref[...], wn_ref[...], preferred_element_type=f32)
    nf += bn_ref[...]
    acc += jnp.dot(nf, wo[48:64, :], preferred_element_type=f32)
    te = jnp.dot(title_ref[...], wt_ref[...], preferred_element_type=f32)
    te += bt_ref[...]
    acc += jnp.dot(te, wo[64:128, :], preferred_element_type=f32)
    out_ref[...] = acc + bo_ref[...]


@jax.jit
def _tc_dense(cat_g, store_g, pa_g, num_pad, title, WnT, bn2, WtT, bt2,
              WoT, bo2):
    R = 2048
    grid = (_B // R,)
    row_blk = lambda i: (i, 0)
    full = lambda i: (0, 0)
    return pl.pallas_call(
        _dense_body,
        grid=grid,
        in_specs=[
            pl.BlockSpec((R, _E), row_blk),
            pl.BlockSpec((R, _E), row_blk),
            pl.BlockSpec((R, _E), row_blk),
            pl.BlockSpec((R, 8), row_blk),
            pl.BlockSpec((R, 384), row_blk),
            pl.BlockSpec((8, 16), full),
            pl.BlockSpec((1, 16), full),
            pl.BlockSpec((384, 64), full),
            pl.BlockSpec((1, 64), full),
            pl.BlockSpec((128, 128), full),
            pl.BlockSpec((1, 128), full),
        ],
        out_specs=pl.BlockSpec((R, 128), row_blk),
        out_shape=jax.ShapeDtypeStruct((_B, 128), jnp.float32),
        compiler_params=pltpu.CompilerParams(
            dimension_semantics=("arbitrary",),
        ),
    )(cat_g, store_g, pa_g, num_pad, title, WnT, bn2, WtT, bt2, WoT, bo2)


def kernel(category, store, parent_asin, numeric_features, title_embedding,
           cat_table, store_table, pa_table, Wn, bn, Wt, bt, Wo, bo):
    ci = category.astype(jnp.int32)
    si = store.astype(jnp.int32)
    pi = parent_asin.astype(jnp.int32)
    cat_g, store_g, pa_g = _sc_gather3(
        ci, si, pi, cat_table, store_table, pa_table)
    num_pad = jnp.pad(numeric_features, ((0, 0), (0, 5)))
    WnT = jnp.pad(Wn.T, ((0, 5), (0, 0)))
    return _tc_dense(
        cat_g, store_g, pa_g, num_pad, title_embedding,
        WnT, bn.reshape(1, 16), Wt.T, bt.reshape(1, 64),
        Wo.T, bo.reshape(1, 128))
